# baseline (speedup 1.0000x reference)
import functools

import jax
import jax.numpy as jnp
from jax import lax
from jax.experimental import pallas as pl
from jax.experimental.pallas import tpu as pltpu
from jax.experimental.pallas import tpu_sc as plsc

D_MODEL = 64
SCALE = 8.0
NUM_WORKERS = 32
CHUNK_A = 128
CHUNK_B = 72


def _scale_rows(rows_v, n_rows):
    @pl.loop(0, n_rows)
    def _row(r):
        for k in range(D_MODEL // 16):
            sl = pl.ds(k * 16, 16)
            rows_v[r, sl] = rows_v[r, sl] * SCALE


def _emb_body(x_hbm, table_hbm, out_hbm, idx_v, rows_a, rows_b, gsem, *, rows_per_w, seq_len):
    wid = lax.axis_index("s") * 2 + lax.axis_index("c")
    base = wid * rows_per_w

    pltpu.sync_copy(x_hbm.at[pl.ds(base, rows_per_w)], idx_v)

    @pl.loop(0, rows_per_w)
    def _row(r):
        row = base + r
        pltpu.async_copy(
            table_hbm.at[idx_v.at[r, pl.ds(0, CHUNK_A)]], rows_a, gsem
        ).wait()
        _scale_rows(rows_a, CHUNK_A)
        pltpu.sync_copy(rows_a, out_hbm.at[row, pl.ds(0, CHUNK_A)])

        pltpu.async_copy(
            table_hbm.at[idx_v.at[r, pl.ds(CHUNK_A, CHUNK_B)]], rows_b, gsem
        ).wait()
        _scale_rows(rows_b, CHUNK_B)
        pltpu.sync_copy(rows_b, out_hbm.at[row, pl.ds(CHUNK_A, CHUNK_B)])


def kernel(x, table):
    B, L = x.shape
    assert L == CHUNK_A + CHUNK_B and B % NUM_WORKERS == 0
    rows_per_w = B // NUM_WORKERS

    mesh = plsc.VectorSubcoreMesh(core_axis_name="c", subcore_axis_name="s")

    emb = functools.partial(
        pl.kernel,
        out_type=jax.ShapeDtypeStruct((B, L, D_MODEL), jnp.float32),
        mesh=mesh,
        compiler_params=pltpu.CompilerParams(use_tc_tiling_on_sc=False),
        scratch_types=[
            pltpu.VMEM((rows_per_w, L), jnp.int32),
            pltpu.VMEM((CHUNK_A, D_MODEL), jnp.float32),
            pltpu.VMEM((CHUNK_B, D_MODEL), jnp.float32),
            pltpu.SemaphoreType.DMA,
        ],
    )(functools.partial(_emb_body, rows_per_w=rows_per_w, seq_len=L))

    return emb(x, table)
